# Initial kernel scaffold; baseline (speedup 1.0000x reference)
import jax
import jax.numpy as jnp
from jax.experimental import pallas as pl


def kernel(x, edge_index, W0, b0, W1, b1):
    raise NotImplementedError("write your pallas kernel here")


# speedup vs baseline: 8.9610x; 8.9610x over previous
import functools

import jax
import jax.numpy as jnp
from jax import lax
from jax.experimental import pallas as pl
from jax.experimental.pallas import tpu as pltpu
from jax.experimental.pallas import tpu_sc as plsc

N = 10000
E = 320000
D = 128

NC = 2
NS = 16
NW = NC * NS

NP = 10240
CHUNK = 128
CPT = 80
EPT = CPT * CHUNK
E_PAD = NW * EPT

_MESH = plsc.VectorSubcoreMesh(core_axis_name="c", subcore_axis_name="s")


@functools.partial(
    pl.kernel,
    out_type=jax.ShapeDtypeStruct((NC * NP,), jnp.float32),
    mesh=_MESH,
    scratch_types=[
        pltpu.VMEM((CPT, CHUNK), jnp.int32),
        pltpu.VMEM((CHUNK,), jnp.float32),
        pltpu.VMEM_SHARED((NP,), jnp.float32),
    ],
)
def _sc_degree(dst_hbm, zeros_np_hbm, ones_hbm, out_hbm, didx_v, ones_v, acc):
    cid = lax.axis_index("c")
    sid = lax.axis_index("s")
    wid = cid * NS + sid

    @pl.when(sid == 0)
    def _():
        pltpu.sync_copy(zeros_np_hbm, acc)

    pltpu.sync_copy(dst_hbm.at[pl.ds(wid * CPT, CPT)], didx_v)
    pltpu.sync_copy(ones_hbm, ones_v)
    plsc.subcore_barrier()

    def body(t, _):
        pltpu.sync_copy(ones_v, acc.at[didx_v.at[t]], add=True)
        return ()

    lax.fori_loop(0, CPT, body, (), unroll=False)
    plsc.subcore_barrier()

    @pl.when(sid == 0)
    def _():
        pltpu.sync_copy(acc, out_hbm.at[pl.ds(cid * NP, NP)])


@functools.partial(
    pl.kernel,
    out_type=jax.ShapeDtypeStruct((NC * NP, D), jnp.float32),
    mesh=_MESH,
    scratch_types=[
        pltpu.VMEM((CPT, CHUNK), jnp.int32),
        pltpu.VMEM((CPT, CHUNK), jnp.int32),
        pltpu.VMEM((CHUNK, D), jnp.float32),
        pltpu.VMEM_SHARED((NP, D), jnp.float32),
        pltpu.SemaphoreType.DMA,
    ],
)
def _sc_aggregate(g_hbm, src_hbm, dst_hbm, zeros_hbm, out_hbm,
                  sidx_v, didx_v, rows_v, acc, sem):
    cid = lax.axis_index("c")
    sid = lax.axis_index("s")
    wid = cid * NS + sid
    rows_per_tile = NP // NS

    pltpu.sync_copy(zeros_hbm, acc.at[pl.ds(sid * rows_per_tile, rows_per_tile)])
    pltpu.sync_copy(src_hbm.at[pl.ds(wid * CPT, CPT)], sidx_v)
    pltpu.sync_copy(dst_hbm.at[pl.ds(wid * CPT, CPT)], didx_v)
    plsc.subcore_barrier()

    def body(t, _):
        pltpu.async_copy(g_hbm.at[sidx_v.at[t]], rows_v, sem).wait()
        pltpu.sync_copy(rows_v, acc.at[didx_v.at[t]], add=True)
        return ()

    lax.fori_loop(0, CPT, body, (), unroll=False)
    plsc.subcore_barrier()

    pltpu.sync_copy(
        acc.at[pl.ds(sid * rows_per_tile, rows_per_tile)],
        out_hbm.at[pl.ds(cid * NP + sid * rows_per_tile, rows_per_tile)],
    )


BLK = 512


def _dinv(d0, d1):
    return lax.rsqrt(d0 + d1 + 1.0)


def _tc_lin_body(x_ref, w_ref, d0_ref, d1_ref, o_ref):
    dinv = _dinv(d0_ref[...], d1_ref[...])
    o_ref[...] = dinv * jnp.dot(x_ref[...], w_ref[...],
                                preferred_element_type=jnp.float32)


def _tc_mid_body(p0_ref, p1_ref, g0_ref, d0_ref, d1_ref, b_ref, w_ref, o_ref):
    dinv = _dinv(d0_ref[...], d1_ref[...])
    h = dinv * (p0_ref[...] + p1_ref[...] + g0_ref[...]) + b_ref[...]
    h = jnp.maximum(h, 0.0)
    nrm = jnp.sqrt(jnp.sum(h * h, axis=1, keepdims=True))
    h = h / jnp.maximum(nrm, 1e-12)
    o_ref[...] = dinv * jnp.dot(h, w_ref[...],
                                preferred_element_type=jnp.float32)


def _tc_fin_body(q0_ref, q1_ref, g1_ref, d0_ref, d1_ref, b_ref, o_ref):
    dinv = _dinv(d0_ref[...], d1_ref[...])
    o_ref[...] = dinv * (q0_ref[...] + q1_ref[...] + g1_ref[...]) + b_ref[...]


def _row_spec():
    return pl.BlockSpec((BLK, D), lambda i: (i, 0))


def _col_spec():
    return pl.BlockSpec((BLK, 1), lambda i: (i, 0))


def _full_spec(shape):
    return pl.BlockSpec(shape, lambda i: (0,) * len(shape))


_GRID = (NP // BLK,)
_ROW_OUT = jax.ShapeDtypeStruct((NP, D), jnp.float32)


def _tc_lin(x, w, d0, d1):
    return pl.pallas_call(
        _tc_lin_body,
        grid=_GRID,
        in_specs=[_row_spec(), _full_spec((D, D)), _col_spec(), _col_spec()],
        out_specs=_row_spec(),
        out_shape=_ROW_OUT,
    )(x, w, d0, d1)


def _tc_mid(p0, p1, g0, d0, d1, b, w):
    return pl.pallas_call(
        _tc_mid_body,
        grid=_GRID,
        in_specs=[_row_spec(), _row_spec(), _row_spec(), _col_spec(),
                  _col_spec(), _full_spec((1, D)), _full_spec((D, D))],
        out_specs=_row_spec(),
        out_shape=_ROW_OUT,
    )(p0, p1, g0, d0, d1, b, w)


def _tc_fin(q0, q1, g1, d0, d1, b):
    return pl.pallas_call(
        _tc_fin_body,
        grid=_GRID,
        in_specs=[_row_spec(), _row_spec(), _row_spec(), _col_spec(),
                  _col_spec(), _full_spec((1, D))],
        out_specs=_row_spec(),
        out_shape=_ROW_OUT,
    )(q0, q1, g1, d0, d1, b)


@jax.jit
def kernel(x, edge_index, W0, b0, W1, b1):
    src = edge_index[0]
    dst = edge_index[1]
    npad = E_PAD - E
    src_p = jnp.concatenate([src, jnp.zeros((npad,), jnp.int32)])
    dst_p = jnp.concatenate([dst, jnp.full((npad,), N, jnp.int32)])
    src2d = src_p.reshape(NW * CPT, CHUNK)
    dst2d = dst_p.reshape(NW * CPT, CHUNK)

    x_pad = jnp.concatenate([x, jnp.zeros((NP - N, D), jnp.float32)])
    zeros_np = jnp.zeros((NP,), jnp.float32)
    ones_chunk = jnp.ones((CHUNK,), jnp.float32)
    zeros_rows = jnp.zeros((NP // NS, D), jnp.float32)

    deg2 = _sc_degree(dst2d, zeros_np, ones_chunk)
    d0 = deg2[:NP].reshape(NP, 1)
    d1 = deg2[NP:].reshape(NP, 1)

    b0_2d = b0.reshape(1, D)
    b1_2d = b1.reshape(1, D)

    g0 = _tc_lin(x_pad, W0, d0, d1)
    parts0 = _sc_aggregate(g0, src2d, dst2d, zeros_rows)
    g1 = _tc_mid(parts0[:NP], parts0[NP:], g0, d0, d1, b0_2d, W1)
    parts1 = _sc_aggregate(g1, src2d, dst2d, zeros_rows)
    out = _tc_fin(parts1[:NP], parts1[NP:], g1, d0, d1, b1_2d)
    return out[:N]


# speedup vs baseline: 10.0659x; 1.1233x over previous
import functools

import jax
import jax.numpy as jnp
from jax import lax
from jax.experimental import pallas as pl
from jax.experimental.pallas import tpu as pltpu
from jax.experimental.pallas import tpu_sc as plsc

N = 10000
E = 320000
D = 128

NC = 2
NS = 16
NW = NC * NS

NP = 10240
CHUNK = 128
CPT = 80
EPT = CPT * CHUNK
E_PAD = NW * EPT

_MESH = plsc.VectorSubcoreMesh(core_axis_name="c", subcore_axis_name="s")


@functools.partial(
    pl.kernel,
    out_type=jax.ShapeDtypeStruct((NC * NP,), jnp.float32),
    mesh=_MESH,
    scratch_types=[
        pltpu.VMEM((CPT, CHUNK), jnp.int32),
        pltpu.VMEM((CHUNK,), jnp.float32),
        pltpu.VMEM_SHARED((NP,), jnp.float32),
    ],
)
def _sc_degree(dst_hbm, zeros_np_hbm, ones_hbm, out_hbm, didx_v, ones_v, acc):
    cid = lax.axis_index("c")
    sid = lax.axis_index("s")
    wid = cid * NS + sid

    @pl.when(sid == 0)
    def _():
        pltpu.sync_copy(zeros_np_hbm, acc)

    pltpu.sync_copy(dst_hbm.at[pl.ds(wid * CPT, CPT)], didx_v)
    pltpu.sync_copy(ones_hbm, ones_v)
    plsc.subcore_barrier()

    def body(t, _):
        pltpu.sync_copy(ones_v, acc.at[didx_v.at[t]], add=True)
        return ()

    lax.fori_loop(0, CPT, body, (), unroll=False)
    plsc.subcore_barrier()

    @pl.when(sid == 0)
    def _():
        pltpu.sync_copy(acc, out_hbm.at[pl.ds(cid * NP, NP)])


@functools.partial(
    pl.kernel,
    out_type=jax.ShapeDtypeStruct((NC * NP, D), jnp.float32),
    mesh=_MESH,
    scratch_types=[
        pltpu.VMEM((CPT // 2, CHUNK), jnp.int32),
        pltpu.VMEM((CPT // 2, CHUNK), jnp.int32),
        pltpu.VMEM((CHUNK, D), jnp.float32),
        pltpu.VMEM((CHUNK, D), jnp.float32),
        pltpu.VMEM_SHARED((NP, D), jnp.float32),
        pltpu.SemaphoreType.DMA,
        pltpu.SemaphoreType.DMA,
    ],
)
def _sc_aggregate(g_hbm, src_hbm, dst_hbm, zeros_hbm, out_hbm,
                  sidx_v, didx_v, rows0_v, rows1_v, acc, sem0, sem1):
    cid = lax.axis_index("c")
    sid = lax.axis_index("s")
    wid = cid * NS + sid
    rows_per_tile = NP // NS

    pltpu.sync_copy(zeros_hbm, acc.at[pl.ds(sid * rows_per_tile, rows_per_tile)])
    plsc.subcore_barrier()

    bufs = (rows0_v, rows1_v)
    sems = (sem0, sem1)
    H = CPT // 2
    for phase in range(2):
        pltpu.sync_copy(src_hbm.at[pl.ds(wid * CPT + phase * H, H)], sidx_v)
        pltpu.sync_copy(dst_hbm.at[pl.ds(wid * CPT + phase * H, H)], didx_v)
        pltpu.async_copy(g_hbm.at[sidx_v.at[0]], rows0_v, sem0)
        pltpu.async_copy(g_hbm.at[sidx_v.at[1]], rows1_v, sem1)

        def body(i, _):
            for b in range(2):
                t = 2 * i + b
                pltpu.make_async_copy(
                    g_hbm.at[sidx_v.at[t]], bufs[b], sems[b]).wait()
                pltpu.sync_copy(bufs[b], acc.at[didx_v.at[t]], add=True)

                @pl.when(t + 2 < H)
                def _():
                    pltpu.async_copy(g_hbm.at[sidx_v.at[t + 2]], bufs[b], sems[b])
            return ()

        lax.fori_loop(0, H // 2, body, (), unroll=False)
    plsc.subcore_barrier()

    pltpu.sync_copy(
        acc.at[pl.ds(sid * rows_per_tile, rows_per_tile)],
        out_hbm.at[pl.ds(cid * NP + sid * rows_per_tile, rows_per_tile)],
    )


BLK = 512


def _dinv(d0, d1):
    return lax.rsqrt(d0 + d1 + 1.0)


def _tc_lin_body(x_ref, w_ref, d0_ref, d1_ref, o_ref):
    dinv = _dinv(d0_ref[...], d1_ref[...])
    o_ref[...] = dinv * jnp.dot(x_ref[...], w_ref[...],
                                preferred_element_type=jnp.float32)


def _tc_mid_body(p0_ref, p1_ref, g0_ref, d0_ref, d1_ref, b_ref, w_ref, o_ref):
    dinv = _dinv(d0_ref[...], d1_ref[...])
    h = dinv * (p0_ref[...] + p1_ref[...] + g0_ref[...]) + b_ref[...]
    h = jnp.maximum(h, 0.0)
    nrm = jnp.sqrt(jnp.sum(h * h, axis=1, keepdims=True))
    h = h / jnp.maximum(nrm, 1e-12)
    o_ref[...] = dinv * jnp.dot(h, w_ref[...],
                                preferred_element_type=jnp.float32)


def _tc_fin_body(q0_ref, q1_ref, g1_ref, d0_ref, d1_ref, b_ref, o_ref):
    dinv = _dinv(d0_ref[...], d1_ref[...])
    o_ref[...] = dinv * (q0_ref[...] + q1_ref[...] + g1_ref[...]) + b_ref[...]


def _row_spec():
    return pl.BlockSpec((BLK, D), lambda i: (i, 0))


def _col_spec():
    return pl.BlockSpec((BLK, 1), lambda i: (i, 0))


def _full_spec(shape):
    return pl.BlockSpec(shape, lambda i: (0,) * len(shape))


_GRID = (NP // BLK,)
_ROW_OUT = jax.ShapeDtypeStruct((NP, D), jnp.float32)


def _tc_lin(x, w, d0, d1):
    return pl.pallas_call(
        _tc_lin_body,
        grid=_GRID,
        in_specs=[_row_spec(), _full_spec((D, D)), _col_spec(), _col_spec()],
        out_specs=_row_spec(),
        out_shape=_ROW_OUT,
    )(x, w, d0, d1)


def _tc_mid(p0, p1, g0, d0, d1, b, w):
    return pl.pallas_call(
        _tc_mid_body,
        grid=_GRID,
        in_specs=[_row_spec(), _row_spec(), _row_spec(), _col_spec(),
                  _col_spec(), _full_spec((1, D)), _full_spec((D, D))],
        out_specs=_row_spec(),
        out_shape=_ROW_OUT,
    )(p0, p1, g0, d0, d1, b, w)


def _tc_fin(q0, q1, g1, d0, d1, b):
    return pl.pallas_call(
        _tc_fin_body,
        grid=_GRID,
        in_specs=[_row_spec(), _row_spec(), _row_spec(), _col_spec(),
                  _col_spec(), _full_spec((1, D))],
        out_specs=_row_spec(),
        out_shape=_ROW_OUT,
    )(q0, q1, g1, d0, d1, b)


@jax.jit
def kernel(x, edge_index, W0, b0, W1, b1):
    src = edge_index[0]
    dst = edge_index[1]
    npad = E_PAD - E
    pad_dst = N + (jnp.arange(npad, dtype=jnp.int32) % (NP - N))
    src_p = jnp.concatenate([src, jnp.zeros((npad,), jnp.int32)])
    dst_p = jnp.concatenate([dst, pad_dst])
    src2d = src_p.reshape(NW * CPT, CHUNK)
    dst2d = dst_p.reshape(NW * CPT, CHUNK)

    x_pad = jnp.concatenate([x, jnp.zeros((NP - N, D), jnp.float32)])
    zeros_np = jnp.zeros((NP,), jnp.float32)
    ones_chunk = jnp.ones((CHUNK,), jnp.float32)
    zeros_rows = jnp.zeros((NP // NS, D), jnp.float32)

    deg2 = _sc_degree(dst2d, zeros_np, ones_chunk)
    d0 = deg2[:NP].reshape(NP, 1)
    d1 = deg2[NP:].reshape(NP, 1)

    b0_2d = b0.reshape(1, D)
    b1_2d = b1.reshape(1, D)

    g0 = _tc_lin(x_pad, W0, d0, d1)
    parts0 = _sc_aggregate(g0, src2d, dst2d, zeros_rows)
    g1 = _tc_mid(parts0[:NP], parts0[NP:], g0, d0, d1, b0_2d, W1)
    parts1 = _sc_aggregate(g1, src2d, dst2d, zeros_rows)
    out = _tc_fin(parts1[:NP], parts1[NP:], g1, d0, d1, b1_2d)
    return out[:N]


# speedup vs baseline: 11.2750x; 1.1201x over previous
import functools

import jax
import jax.numpy as jnp
from jax import lax
from jax.experimental import pallas as pl
from jax.experimental.pallas import tpu as pltpu
from jax.experimental.pallas import tpu_sc as plsc

N = 10000
E = 320000
D = 128

NC = 2
NS = 16
NW = NC * NS

NP = 10240
CHUNK = 128
CPT = 80
EPT = CPT * CHUNK
E_PAD = NW * EPT
H = 40
CPT_C0 = 120
CPT_C1 = 40

_MESH = plsc.VectorSubcoreMesh(core_axis_name="c", subcore_axis_name="s")


@functools.partial(
    pl.kernel,
    out_type=jax.ShapeDtypeStruct((NC * NP,), jnp.float32),
    mesh=_MESH,
    scratch_types=[
        pltpu.VMEM((CPT, CHUNK), jnp.int32),
        pltpu.VMEM((CHUNK,), jnp.float32),
        pltpu.VMEM_SHARED((NP,), jnp.float32),
    ],
)
def _sc_degree(dst_hbm, zeros_np_hbm, ones_hbm, out_hbm, didx_v, ones_v, acc):
    cid = lax.axis_index("c")
    sid = lax.axis_index("s")
    wid = cid * NS + sid

    @pl.when(sid == 0)
    def _():
        pltpu.sync_copy(zeros_np_hbm, acc)

    pltpu.sync_copy(dst_hbm.at[pl.ds(wid * CPT, CPT)], didx_v)
    pltpu.sync_copy(ones_hbm, ones_v)
    plsc.subcore_barrier()

    def body(t, _):
        pltpu.sync_copy(ones_v, acc.at[didx_v.at[t]], add=True)
        return ()

    lax.fori_loop(0, CPT, body, (), unroll=False)
    plsc.subcore_barrier()

    @pl.when(sid == 0)
    def _():
        pltpu.sync_copy(acc, out_hbm.at[pl.ds(cid * NP, NP)])


@functools.partial(
    pl.kernel,
    out_type=jax.ShapeDtypeStruct((NC * NP, D), jnp.float32),
    mesh=_MESH,
    scratch_types=[
        pltpu.VMEM((H, CHUNK), jnp.int32),
        pltpu.VMEM((H, CHUNK), jnp.int32),
        pltpu.VMEM((CHUNK, D), jnp.float32),
        pltpu.VMEM((CHUNK, D), jnp.float32),
        pltpu.VMEM_SHARED((NP, D), jnp.float32),
        pltpu.SemaphoreType.DMA,
        pltpu.SemaphoreType.DMA,
    ],
)
def _sc_aggregate(g_hbm, src_hbm, dst_hbm, zeros_hbm, out_hbm,
                  sidx_v, didx_v, rows0_v, rows1_v, acc, sem0, sem1):
    cid = lax.axis_index("c")
    sid = lax.axis_index("s")
    rows_per_tile = NP // NS

    pltpu.sync_copy(zeros_hbm, acc.at[pl.ds(sid * rows_per_tile, rows_per_tile)])
    plsc.subcore_barrier()

    bufs = (rows0_v, rows1_v)
    sems = (sem0, sem1)

    def run(base, nchunks):
        for phase in range(nchunks // H):
            pltpu.sync_copy(src_hbm.at[pl.ds(base + phase * H, H)], sidx_v)
            pltpu.sync_copy(dst_hbm.at[pl.ds(base + phase * H, H)], didx_v)
            pltpu.async_copy(g_hbm.at[sidx_v.at[0]], rows0_v, sem0)
            pltpu.async_copy(g_hbm.at[sidx_v.at[1]], rows1_v, sem1)

            def body(i, _):
                for b in range(2):
                    t = 2 * i + b
                    pltpu.make_async_copy(
                        g_hbm.at[sidx_v.at[t]], bufs[b], sems[b]).wait()
                    pltpu.sync_copy(bufs[b], acc.at[didx_v.at[t]], add=True)

                    @pl.when(t + 2 < H)
                    def _():
                        pltpu.async_copy(
                            g_hbm.at[sidx_v.at[t + 2]], bufs[b], sems[b])
                return ()

            lax.fori_loop(0, H // 2, body, (), unroll=False)

    @pl.when(cid == 0)
    def _():
        run(sid * (CPT_C0 + CPT_C1), CPT_C0)

    @pl.when(cid == 1)
    def _():
        run(sid * (CPT_C0 + CPT_C1) + CPT_C0, CPT_C1)

    plsc.subcore_barrier()

    pltpu.sync_copy(
        acc.at[pl.ds(sid * rows_per_tile, rows_per_tile)],
        out_hbm.at[pl.ds(cid * NP + sid * rows_per_tile, rows_per_tile)],
    )


BLK = 512


def _dinv(d0, d1):
    return lax.rsqrt(d0 + d1 + 1.0)


def _tc_lin_body(x_ref, w_ref, d0_ref, d1_ref, o_ref):
    dinv = _dinv(d0_ref[...], d1_ref[...])
    o_ref[...] = dinv * jnp.dot(x_ref[...], w_ref[...],
                                preferred_element_type=jnp.float32)


def _tc_mid_body(p0_ref, p1_ref, g0_ref, d0_ref, d1_ref, b_ref, w_ref, o_ref):
    dinv = _dinv(d0_ref[...], d1_ref[...])
    h = dinv * (p0_ref[...] + p1_ref[...] + g0_ref[...]) + b_ref[...]
    h = jnp.maximum(h, 0.0)
    nrm = jnp.sqrt(jnp.sum(h * h, axis=1, keepdims=True))
    h = h / jnp.maximum(nrm, 1e-12)
    o_ref[...] = dinv * jnp.dot(h, w_ref[...],
                                preferred_element_type=jnp.float32)


def _tc_fin_body(q0_ref, q1_ref, g1_ref, d0_ref, d1_ref, b_ref, o_ref):
    dinv = _dinv(d0_ref[...], d1_ref[...])
    o_ref[...] = dinv * (q0_ref[...] + q1_ref[...] + g1_ref[...]) + b_ref[...]


def _row_spec():
    return pl.BlockSpec((BLK, D), lambda i: (i, 0))


def _col_spec():
    return pl.BlockSpec((BLK, 1), lambda i: (i, 0))


def _full_spec(shape):
    return pl.BlockSpec(shape, lambda i: (0,) * len(shape))


_GRID = (NP // BLK,)
_ROW_OUT = jax.ShapeDtypeStruct((NP, D), jnp.float32)


def _tc_lin(x, w, d0, d1):
    return pl.pallas_call(
        _tc_lin_body,
        grid=_GRID,
        in_specs=[_row_spec(), _full_spec((D, D)), _col_spec(), _col_spec()],
        out_specs=_row_spec(),
        out_shape=_ROW_OUT,
    )(x, w, d0, d1)


def _tc_mid(p0, p1, g0, d0, d1, b, w):
    return pl.pallas_call(
        _tc_mid_body,
        grid=_GRID,
        in_specs=[_row_spec(), _row_spec(), _row_spec(), _col_spec(),
                  _col_spec(), _full_spec((1, D)), _full_spec((D, D))],
        out_specs=_row_spec(),
        out_shape=_ROW_OUT,
    )(p0, p1, g0, d0, d1, b, w)


def _tc_fin(q0, q1, g1, d0, d1, b):
    return pl.pallas_call(
        _tc_fin_body,
        grid=_GRID,
        in_specs=[_row_spec(), _row_spec(), _row_spec(), _col_spec(),
                  _col_spec(), _full_spec((1, D))],
        out_specs=_row_spec(),
        out_shape=_ROW_OUT,
    )(q0, q1, g1, d0, d1, b)


@jax.jit
def kernel(x, edge_index, W0, b0, W1, b1):
    src = edge_index[0]
    dst = edge_index[1]
    npad = E_PAD - E
    pad_dst = N + (jnp.arange(npad, dtype=jnp.int32) % (NP - N))
    src_p = jnp.concatenate([src, jnp.zeros((npad,), jnp.int32)])
    dst_p = jnp.concatenate([dst, pad_dst])
    src2d = src_p.reshape(NW * CPT, CHUNK)
    dst2d = dst_p.reshape(NW * CPT, CHUNK)

    x_pad = jnp.concatenate([x, jnp.zeros((NP - N, D), jnp.float32)])
    zeros_np = jnp.zeros((NP,), jnp.float32)
    ones_chunk = jnp.ones((CHUNK,), jnp.float32)
    zeros_rows = jnp.zeros((NP // NS, D), jnp.float32)

    deg2 = _sc_degree(dst2d, zeros_np, ones_chunk)
    d0 = deg2[:NP].reshape(NP, 1)
    d1 = deg2[NP:].reshape(NP, 1)

    b0_2d = b0.reshape(1, D)
    b1_2d = b1.reshape(1, D)

    g0 = _tc_lin(x_pad, W0, d0, d1)
    parts0 = _sc_aggregate(g0, src2d, dst2d, zeros_rows)
    g1 = _tc_mid(parts0[:NP], parts0[NP:], g0, d0, d1, b0_2d, W1)
    parts1 = _sc_aggregate(g1, src2d, dst2d, zeros_rows)
    out = _tc_fin(parts1[:NP], parts1[NP:], g1, d0, d1, b1_2d)
    return out[:N]


# speedup vs baseline: 28.3018x; 2.5101x over previous
import functools

import jax
import jax.numpy as jnp
from jax import lax
from jax.experimental import pallas as pl
from jax.experimental.pallas import tpu as pltpu
from jax.experimental.pallas import tpu_sc as plsc

N = 10000
E = 320000
D = 128

NC = 2
NS = 16
NW = NC * NS

NP = 10240
CHUNK = 128
CPT = 80
EPT = CPT * CHUNK
E_PAD = NW * EPT
H = 40
CPT_C0 = 80
CPT_C1 = 80

_MESH = plsc.VectorSubcoreMesh(core_axis_name="c", subcore_axis_name="s")


@functools.partial(
    pl.kernel,
    out_type=jax.ShapeDtypeStruct((NC * NP,), jnp.float32),
    mesh=_MESH,
    scratch_types=[
        pltpu.VMEM((CPT, CHUNK), jnp.int32),
        pltpu.VMEM((CHUNK,), jnp.float32),
        pltpu.VMEM_SHARED((NP,), jnp.float32),
    ],
)
def _sc_degree(dst_hbm, zeros_np_hbm, ones_hbm, out_hbm, didx_v, ones_v, acc):
    cid = lax.axis_index("c")
    sid = lax.axis_index("s")
    wid = cid * NS + sid

    @pl.when(sid == 0)
    def _():
        pltpu.sync_copy(zeros_np_hbm, acc)

    pltpu.sync_copy(dst_hbm.at[pl.ds(wid * CPT, CPT)], didx_v)
    pltpu.sync_copy(ones_hbm, ones_v)
    plsc.subcore_barrier()

    def body(t, _):
        pltpu.sync_copy(ones_v, acc.at[didx_v.at[t]], add=True)
        return ()

    lax.fori_loop(0, CPT, body, (), unroll=False)
    plsc.subcore_barrier()

    @pl.when(sid == 0)
    def _():
        pltpu.sync_copy(acc, out_hbm.at[pl.ds(cid * NP, NP)])


@functools.partial(
    pl.kernel,
    out_type=jax.ShapeDtypeStruct((NC * NP, D), jnp.float32),
    mesh=_MESH,
    scratch_types=[
        pltpu.VMEM((H, CHUNK), jnp.int32),
        pltpu.VMEM((H, CHUNK), jnp.int32),
        pltpu.VMEM((CHUNK, D), jnp.float32),
        pltpu.VMEM((CHUNK, D), jnp.float32),
        pltpu.VMEM_SHARED((NP, D), jnp.float32),
        pltpu.SemaphoreType.DMA,
        pltpu.SemaphoreType.DMA,
    ],
)
def _sc_aggregate(g_hbm, src_hbm, dst_hbm, zeros_hbm, out_hbm,
                  sidx_v, didx_v, rows0_v, rows1_v, acc, sem0, sem1):
    cid = lax.axis_index("c")
    sid = lax.axis_index("s")
    rows_per_tile = NP // NS

    pltpu.sync_copy(zeros_hbm, acc.at[pl.ds(sid * rows_per_tile, rows_per_tile)])
    plsc.subcore_barrier()

    bufs = (rows0_v, rows1_v)
    sems = (sem0, sem1)

    def run(base, nchunks):
        for phase in range(nchunks // H):
            pltpu.sync_copy(src_hbm.at[pl.ds(base + phase * H, H)], sidx_v)
            pltpu.sync_copy(dst_hbm.at[pl.ds(base + phase * H, H)], didx_v)
            pltpu.async_copy(g_hbm.at[sidx_v.at[0]], rows0_v, sem0)
            pltpu.async_copy(g_hbm.at[sidx_v.at[1]], rows1_v, sem1)

            def body(i, _):
                for b in range(2):
                    t = 2 * i + b
                    pltpu.make_async_copy(
                        g_hbm.at[sidx_v.at[t]], bufs[b], sems[b]).wait()
                    pltpu.sync_copy(bufs[b], acc.at[didx_v.at[t]], add=True)

                    @pl.when(t + 2 < H)
                    def _():
                        pltpu.async_copy(
                            g_hbm.at[sidx_v.at[t + 2]], bufs[b], sems[b])
                return ()

            lax.fori_loop(0, H // 2, body, (), unroll=False)

    @pl.when(cid == 0)
    def _():
        run(sid * (CPT_C0 + CPT_C1), CPT_C0)

    @pl.when(cid == 1)
    def _():
        run(sid * (CPT_C0 + CPT_C1) + CPT_C0, CPT_C1)

    plsc.subcore_barrier()

    pltpu.sync_copy(
        acc.at[pl.ds(sid * rows_per_tile, rows_per_tile)],
        out_hbm.at[pl.ds(cid * NP + sid * rows_per_tile, rows_per_tile)],
    )


BLK = 512


def _dinv(d0, d1):
    return lax.rsqrt(d0 + d1 + 1.0)


def _tc_lin_body(x_ref, w_ref, d0_ref, d1_ref, o_ref):
    dinv = _dinv(d0_ref[...], d1_ref[...])
    o_ref[...] = dinv * jnp.dot(x_ref[...], w_ref[...],
                                preferred_element_type=jnp.float32)


def _tc_mid_body(p0_ref, p1_ref, g0_ref, d0_ref, d1_ref, b_ref, w_ref, o_ref):
    dinv = _dinv(d0_ref[...], d1_ref[...])
    h = dinv * (p0_ref[...] + p1_ref[...] + g0_ref[...]) + b_ref[...]
    h = jnp.maximum(h, 0.0)
    nrm = jnp.sqrt(jnp.sum(h * h, axis=1, keepdims=True))
    h = h / jnp.maximum(nrm, 1e-12)
    o_ref[...] = dinv * jnp.dot(h, w_ref[...],
                                preferred_element_type=jnp.float32)


def _tc_fin_body(q0_ref, q1_ref, g1_ref, d0_ref, d1_ref, b_ref, o_ref):
    dinv = _dinv(d0_ref[...], d1_ref[...])
    o_ref[...] = dinv * (q0_ref[...] + q1_ref[...] + g1_ref[...]) + b_ref[...]


def _row_spec():
    return pl.BlockSpec((BLK, D), lambda i: (i, 0))


def _col_spec():
    return pl.BlockSpec((BLK, 1), lambda i: (i, 0))


def _full_spec(shape):
    return pl.BlockSpec(shape, lambda i: (0,) * len(shape))


_GRID = (NP // BLK,)
_ROW_OUT = jax.ShapeDtypeStruct((NP, D), jnp.float32)


def _tc_lin(x, w, d0, d1):
    return pl.pallas_call(
        _tc_lin_body,
        grid=_GRID,
        in_specs=[_row_spec(), _full_spec((D, D)), _col_spec(), _col_spec()],
        out_specs=_row_spec(),
        out_shape=_ROW_OUT,
    )(x, w, d0, d1)


def _tc_mid(p0, p1, g0, d0, d1, b, w):
    return pl.pallas_call(
        _tc_mid_body,
        grid=_GRID,
        in_specs=[_row_spec(), _row_spec(), _row_spec(), _col_spec(),
                  _col_spec(), _full_spec((1, D)), _full_spec((D, D))],
        out_specs=_row_spec(),
        out_shape=_ROW_OUT,
    )(p0, p1, g0, d0, d1, b, w)


def _tc_fin(q0, q1, g1, d0, d1, b):
    return pl.pallas_call(
        _tc_fin_body,
        grid=_GRID,
        in_specs=[_row_spec(), _row_spec(), _row_spec(), _col_spec(),
                  _col_spec(), _full_spec((1, D))],
        out_specs=_row_spec(),
        out_shape=_ROW_OUT,
    )(q0, q1, g1, d0, d1, b)


@jax.jit
def kernel(x, edge_index, W0, b0, W1, b1):
    src = edge_index[0]
    dst = edge_index[1]
    npad = E_PAD - E
    pad_iota = jnp.arange(npad, dtype=jnp.int32)
    pad_dst = N + pad_iota % (NP - N)
    pad_src = pad_iota % N
    src_p = jnp.concatenate([src, pad_src])
    dst_p = jnp.concatenate([dst, pad_dst])
    src2d = src_p.reshape(NW * CPT, CHUNK)
    dst2d = dst_p.reshape(NW * CPT, CHUNK)

    x_pad = jnp.concatenate([x, jnp.zeros((NP - N, D), jnp.float32)])
    zeros_np = jnp.zeros((NP,), jnp.float32)
    ones_chunk = jnp.ones((CHUNK,), jnp.float32)
    zeros_rows = jnp.zeros((NP // NS, D), jnp.float32)

    deg2 = _sc_degree(dst2d, zeros_np, ones_chunk)
    d0 = deg2[:NP].reshape(NP, 1)
    d1 = deg2[NP:].reshape(NP, 1)

    b0_2d = b0.reshape(1, D)
    b1_2d = b1.reshape(1, D)

    g0 = _tc_lin(x_pad, W0, d0, d1)
    parts0 = _sc_aggregate(g0, src2d, dst2d, zeros_rows)
    g1 = _tc_mid(parts0[:NP], parts0[NP:], g0, d0, d1, b0_2d, W1)
    parts1 = _sc_aggregate(g1, src2d, dst2d, zeros_rows)
    out = _tc_fin(parts1[:NP], parts1[NP:], g1, d0, d1, b1_2d)
    return out[:N]
